# baseline (speedup 1.0000x reference)
import functools

import jax
import jax.numpy as jnp
from jax import lax
from jax.experimental import pallas as pl
from jax.experimental.pallas import tpu as pltpu
from jax.experimental.pallas import tpu_sc as plsc

NC = 2
NS = 16
L = 16
NW = NC * NS

B = 16384
D = 32
BPW = B // NW
G = BPW // L

_mesh = plsc.VectorSubcoreMesh(core_axis_name="c", subcore_axis_name="s")


@functools.partial(
    pl.kernel,
    out_type=jax.ShapeDtypeStruct((B,), jnp.float32),
    mesh=_mesh,
    scratch_types=[
        pltpu.VMEM((BPW,), jnp.int32),
        pltpu.VMEM((BPW,), jnp.int32),
        pltpu.VMEM((BPW, D), jnp.float32),
        pltpu.VMEM((BPW, D), jnp.float32),
        pltpu.VMEM((2 * D, L), jnp.float32),
        pltpu.VMEM((L,), jnp.float32),
        pltpu.VMEM((BPW,), jnp.float32),
        pltpu.SemaphoreType.DMA,
        pltpu.SemaphoreType.DMA,
    ],
    compiler_params=pltpu.CompilerParams(
        needs_layout_passes=False, use_tc_tiling_on_sc=False),
)
def _ncf_sc(uids, iids, utab, itab, wb, bb, out,
            uidx_v, iidx_v, urows, irows, w_v, b_v, out_v, sem_u, sem_i):
    wid = lax.axis_index("s") * NC + lax.axis_index("c")
    base = wid * BPW

    pltpu.sync_copy(uids.at[pl.ds(base, BPW)], uidx_v)
    pltpu.sync_copy(iids.at[pl.ds(base, BPW)], iidx_v)
    cu = pltpu.async_copy(utab.at[uidx_v], urows, sem_u)
    ci = pltpu.async_copy(itab.at[iidx_v], irows, sem_i)
    pltpu.sync_copy(wb, w_v)
    pltpu.sync_copy(bb, b_v)
    cu.wait()
    ci.wait()

    def group(g, carry):
        rowi = g * L + lax.iota(jnp.int32, L)
        flat = rowi * D
        acc = b_v[...]
        for d in range(D):
            col = jnp.full((L,), d, jnp.int32)
            acc = acc + plsc.load_gather(urows, [rowi, col]) * w_v[d, :]
        for d in range(D):
            col = jnp.full((L,), d, jnp.int32)
            acc = acc + plsc.load_gather(irows, [rowi, col]) * w_v[D + d, :]
        plsc.store_scatter(out_v, [rowi], acc)
        return carry

    lax.fori_loop(0, G, group, 0)
    pltpu.sync_copy(out_v, out.at[pl.ds(base, BPW)])


def kernel(user_ids, item_ids, user_table, item_table, W, b):
    wb = jnp.broadcast_to(W.reshape(2 * D, 1), (2 * D, L))
    bb = jnp.broadcast_to(b, (L,))
    out = _ncf_sc(user_ids, item_ids, user_table, item_table, wb, bb)
    return out.reshape(B, 1)
